# Initial kernel scaffold; baseline (speedup 1.0000x reference)
import jax
import jax.numpy as jnp
from jax.experimental import pallas as pl


def kernel(x, edge_attr, params, edge_index, batch):
    raise NotImplementedError("write your pallas kernel here")


# speedup vs baseline: 2.3452x; 2.3452x over previous
import functools

import jax
import jax.numpy as jnp
from jax import lax
from jax.experimental import pallas as pl
from jax.experimental.pallas import tpu as pltpu
from jax.experimental.pallas import tpu_sc as plsc

_N = 10000
_E = 320000
_H = 128
_G = 128
_EDGE_DIM = 16
_DEPTH = 3

_NC = 2
_NS = 16
_NW = _NC * _NS
_EW = _E // _NW
_C = 80
_NCHUNK = _EW // _C
_NP = 10240
_RT = _NP // _NS


def _sc_mesh():
    return plsc.VectorSubcoreMesh(core_axis_name="c", subcore_axis_name="s",
                                  num_cores=_NC, num_subcores=_NS)


@functools.cache
def _sc_gather_kernel():
    @functools.partial(
        pl.kernel,
        out_type=jax.ShapeDtypeStruct((_E, _H), jnp.float32),
        mesh=_sc_mesh(),
        scratch_types=[
            pltpu.VMEM((_C,), jnp.int32),
            pltpu.VMEM((_C, _H), jnp.float32),
            pltpu.SemaphoreType.DMA,
        ],
    )
    def gk(x_hbm, src_hbm, out_hbm, idx_v, rows_v, sem):
        wid = lax.axis_index("s") * _NC + lax.axis_index("c")
        base = wid * _EW

        def body(i, carry):
            off = base + i * _C
            pltpu.sync_copy(src_hbm.at[pl.ds(off, _C)], idx_v)
            pltpu.async_copy(x_hbm.at[idx_v], rows_v, sem).wait()
            pltpu.sync_copy(rows_v, out_hbm.at[pl.ds(off, _C)])
            return carry

        lax.fori_loop(0, _NCHUNK, body, 0)

    return gk


def _sc_gather(xarr, src):
    return _sc_gather_kernel()(xarr, src)


@functools.cache
def _sc_scatter_kernel():
    @functools.partial(
        pl.kernel,
        out_type=jax.ShapeDtypeStruct((2 * _NP, _H), jnp.float32),
        mesh=_sc_mesh(),
        scratch_types=[
            pltpu.VMEM((_C,), jnp.int32),
            pltpu.VMEM((_C, _H), jnp.float32),
            pltpu.VMEM_SHARED((_NP, _H), jnp.float32),
            pltpu.SemaphoreType.DMA,
        ],
    )
    def sk(m_hbm, dst_hbm, zero_hbm, out_hbm, idx_v, buf_v, acc, sem):
        cid = lax.axis_index("c")
        sid = lax.axis_index("s")
        wid = sid * _NC + cid
        row0 = sid * _RT
        pltpu.sync_copy(zero_hbm.at[pl.ds(row0, _RT)], acc.at[pl.ds(row0, _RT)])
        plsc.subcore_barrier()

        base = wid * _EW

        def body(i, carry):
            off = base + i * _C
            pltpu.sync_copy(dst_hbm.at[pl.ds(off, _C)], idx_v)
            pltpu.sync_copy(m_hbm.at[pl.ds(off, _C)], buf_v)
            pltpu.sync_copy(buf_v, acc.at[idx_v], add=True)
            return carry

        lax.fori_loop(0, _NCHUNK, body, 0)
        plsc.subcore_barrier()
        pltpu.sync_copy(acc.at[pl.ds(row0, _RT)],
                        out_hbm.at[pl.ds(cid * _NP + row0, _RT)])

    return sk


def _sc_scatter(m, dst, zeros):
    return _sc_scatter_kernel()(m, dst, zeros)


def _dot(a, b):
    return jnp.dot(a.astype(jnp.bfloat16), b.astype(jnp.bfloat16),
                   preferred_element_type=jnp.float32)


def _dot_hi(a, b):
    return jnp.dot(a, b, preferred_element_type=jnp.float32,
                   precision=lax.Precision.HIGHEST)


def _bn(t, g, b):
    mu = jnp.mean(t, axis=0, keepdims=True)
    var = jnp.mean((t - mu) ** 2, axis=0, keepdims=True)
    return g * (t - mu) * jax.lax.rsqrt(var + 1e-5) + b


def _tc_call(body, out_shapes, *args):
    return pl.pallas_call(
        body,
        out_shape=out_shapes,
    )(*args)


def _k_in_body(x, w0, b0, g0, bb0, w1, b1, g1, bb1, w2, b2, vemb, x1_out):
    t = _dot(x[...], w0[...]) + b0[...]
    t = jax.nn.relu(_bn(t, g0[...], bb0[...]))
    t = _dot(t, w1[...]) + b1[...]
    t = jax.nn.relu(_bn(t, g1[...], bb1[...]))
    t = _dot(t, w2[...]) + b2[...]
    x1_out[...] = t + vemb[...]


_BE = 4000


def _k_msg_body(gx, ea, w, b, m_out):
    ee = _dot(ea[...], w[...]) + b[...]
    m_out[...] = jax.nn.relu(gx[...] + ee)


def _k_msg(gx, ea, w, b):
    grid = _E // _BE
    return pl.pallas_call(
        _k_msg_body,
        grid=(grid,),
        in_specs=[
            pl.BlockSpec((_BE, _H), lambda i: (i, 0)),
            pl.BlockSpec((_BE, _EDGE_DIM), lambda i: (i, 0)),
            pl.BlockSpec((_EDGE_DIM, _H), lambda i: (0, 0)),
            pl.BlockSpec((1, _H), lambda i: (0, 0)),
        ],
        out_specs=pl.BlockSpec((_BE, _H), lambda i: (i, 0)),
        out_shape=jax.ShapeDtypeStruct((_E, _H), jnp.float32),
    )(gx, ea, w, b)


def _k_layer_body(xi, parts, ve, bcol, brow, eps,
                  w1, b1, g1, bb1, w2, b2,
                  vw1, vb1, vg1, vbb1, vw2, vb2, vg2, vbb2, vw3, vb3,
                  h_out, xn_out, ve_out):
    agg = parts[0:_N, :] + parts[_NP:_NP + _N, :]
    t = (1.0 + eps[0, 0]) * xi[...] + agg
    t = _dot(t, w1[...]) + b1[...]
    t = jax.nn.relu(_bn(t, g1[...], bb1[...]))
    t = _dot(t, w2[...]) + b2[...]
    h = t + xi[...]
    h_out[...] = h
    oht = (lax.broadcasted_iota(jnp.int32, (_G, _N), 0) == brow[...]
           ).astype(jnp.float32)
    pooled = _dot_hi(oht, h)
    v = pooled + ve[...]
    v = _dot(v, vw1[...]) + vb1[...]
    v = jax.nn.relu(_bn(v, vg1[...], vbb1[...]))
    v = _dot(v, vw2[...]) + vb2[...]
    v = jax.nn.relu(_bn(v, vg2[...], vbb2[...]))
    v = _dot(v, vw3[...]) + vb3[...]
    ve_new = ve[...] + v
    ve_out[...] = ve_new
    oh = (lax.broadcasted_iota(jnp.int32, (_N, _G), 1) == bcol[...]
          ).astype(jnp.float32)
    xn_out[...] = h + _dot_hi(oh, ve_new)


_BJ = 2000


def _k_join_body(x0, x1, x2, x3, x4, wa, wb, wc, wd, we, b1, y_out):
    y_out[...] = (_dot(x0[...], wa[...])
                  + _dot(x1[...], wb[...])
                  + _dot(x2[...], wc[...])
                  + _dot(x3[...], wd[...])
                  + _dot(x4[...], we[...])
                  + b1[...])


def _k_join(x0, x1, x2, x3, x4, wa, wb, wc, wd, we, b1):
    bs = lambda i: (i, 0)
    return pl.pallas_call(
        _k_join_body,
        grid=(_N // _BJ,),
        in_specs=[pl.BlockSpec((_BJ, _H), bs)] * 5
        + [pl.BlockSpec((_H, _H), lambda i: (0, 0))] * 5
        + [pl.BlockSpec((1, _H), lambda i: (0, 0))],
        out_specs=pl.BlockSpec((_BJ, _H), bs),
        out_shape=jax.ShapeDtypeStruct((_N, _H), jnp.float32),
    )(x0, x1, x2, x3, x4, wa, wb, wc, wd, we, b1)


def _k_out_body(y_in, g1, bb1, w2, b2, g2, bb2, w3, b3, brow,
                nodes_out, graph_out):
    y = jax.nn.relu(_bn(y_in[...], g1[...], bb1[...]))
    y = _dot(y, w2[...]) + b2[...]
    y = jax.nn.relu(_bn(y, g2[...], bb2[...]))
    nodes = _dot(y, w3[...]) + b3[...]
    nodes_out[...] = nodes
    oht = (lax.broadcasted_iota(jnp.int32, (_G, _N), 0) == brow[...]
           ).astype(jnp.float32)
    counts = jnp.maximum(jnp.sum(oht, axis=1, keepdims=True), 1.0)
    graph_out[...] = _dot_hi(oht, nodes) / counts


def _lin_t(p):
    return p["W"].T, p["b"].reshape(1, -1)


def kernel(x, edge_attr, params, edge_index, batch):
    src = edge_index[0]
    dst = edge_index[1]
    bcol = batch.reshape(_N, 1)
    brow = batch.reshape(1, _N)
    zeros = jnp.zeros((_NP, _H), jnp.float32)

    ip = params["in_layer"]
    w0, b0 = _lin_t(ip["lins"][0])
    w1, b1 = _lin_t(ip["lins"][1])
    w2, b2 = _lin_t(ip["lins"][2])
    g0 = ip["bns"][0]["g"].reshape(1, -1)
    bb0 = ip["bns"][0]["b"].reshape(1, -1)
    g1 = ip["bns"][1]["g"].reshape(1, -1)
    bb1 = ip["bns"][1]["b"].reshape(1, -1)
    vemb = params["virtual_embed"].reshape(1, _H)

    x1 = _tc_call(_k_in_body, jax.ShapeDtypeStruct((_N, _H), jnp.float32),
                  x, w0, b0, g0, bb0, w1, b1, g1, bb1, w2, b2, vemb)

    ve = jnp.tile(params["virtual_embed"], (_G, 1))
    xs = [x, x1]
    xi = x1
    for i in range(_DEPTH):
        cp = params["convs"][i]
        vp = params["virtual_layers"][i]
        we_w, we_b = _lin_t(cp["edge"])
        gx = _sc_gather(xi, src)
        m = _k_msg(gx, edge_attr, we_w, we_b)
        parts = _sc_scatter(m, dst, zeros)

        nw1, nb1 = _lin_t(cp["nn_lin1"])
        nw2, nb2 = _lin_t(cp["nn_lin2"])
        ng = cp["nn_bn"]["g"].reshape(1, -1)
        nbb = cp["nn_bn"]["b"].reshape(1, -1)
        vw1, vb1 = _lin_t(vp["lin1"])
        vw2, vb2 = _lin_t(vp["lin2"])
        vw3, vb3 = _lin_t(vp["lin3"])
        vg1 = vp["bn1"]["g"].reshape(1, -1)
        vbb1 = vp["bn1"]["b"].reshape(1, -1)
        vg2 = vp["bn2"]["g"].reshape(1, -1)
        vbb2 = vp["bn2"]["b"].reshape(1, -1)
        eps = cp["eps"].reshape(1, 1)

        h, xn, ve = _tc_call(
            _k_layer_body,
            (jax.ShapeDtypeStruct((_N, _H), jnp.float32),
             jax.ShapeDtypeStruct((_N, _H), jnp.float32),
             jax.ShapeDtypeStruct((_G, _H), jnp.float32)),
            xi, parts, ve, bcol, brow, eps,
            nw1, nb1, ng, nbb, nw2, nb2,
            vw1, vb1, vg1, vbb1, vw2, vb2, vg2, vbb2, vw3, vb3)
        if i < _DEPTH - 1:
            xs.append(xn)
            xi = xn
        else:
            xs.append(h)

    op = params["out_layer"]
    ow = op["lins"][0]["W"]
    ob1 = op["lins"][0]["b"].reshape(1, -1)
    wa = ow[:, 0:_H].T
    wb = ow[:, _H:2 * _H].T
    wc = ow[:, 2 * _H:3 * _H].T
    wd = ow[:, 3 * _H:4 * _H].T
    we = ow[:, 4 * _H:5 * _H].T
    ow2, ob2 = _lin_t(op["lins"][1])
    ow3, ob3 = _lin_t(op["lins"][2])
    og1 = op["bns"][0]["g"].reshape(1, -1)
    obb1 = op["bns"][0]["b"].reshape(1, -1)
    og2 = op["bns"][1]["g"].reshape(1, -1)
    obb2 = op["bns"][1]["b"].reshape(1, -1)

    y = _k_join(xs[0], xs[1], xs[2], xs[3], xs[4], wa, wb, wc, wd, we, ob1)
    nodes, graph = _tc_call(
        _k_out_body,
        (jax.ShapeDtypeStruct((_N, _H), jnp.float32),
         jax.ShapeDtypeStruct((_G, _H), jnp.float32)),
        y, og1, obb1, ow2, ob2, og2, obb2, ow3, ob3, brow)
    return (nodes, graph)


# speedup vs baseline: 2.9724x; 1.2674x over previous
import functools

import jax
import jax.numpy as jnp
from jax import lax
from jax.experimental import pallas as pl
from jax.experimental.pallas import tpu as pltpu
from jax.experimental.pallas import tpu_sc as plsc

_N = 10000
_E = 320000
_H = 128
_G = 128
_EDGE_DIM = 16
_DEPTH = 3

_NC = 2
_NS = 16
_NW = _NC * _NS
_EW = _E // _NW
_C = 80
_NCHUNK = _EW // _C
_NP = 10240
_RT = _NP // _NS


def _sc_mesh():
    return plsc.VectorSubcoreMesh(core_axis_name="c", subcore_axis_name="s",
                                  num_cores=_NC, num_subcores=_NS)


@functools.cache
def _sc_edge_kernel():
    @functools.partial(
        pl.kernel,
        out_type=jax.ShapeDtypeStruct((2 * _NP, _H), jnp.float32),
        mesh=_sc_mesh(),
        scratch_types=[
            pltpu.VMEM((_C,), jnp.int32),
            pltpu.VMEM((_C,), jnp.int32),
            pltpu.VMEM((_C, _H), jnp.float32),
            pltpu.VMEM((_C, _H), jnp.float32),
            pltpu.VMEM_SHARED((_NP, _H), jnp.float32),
            pltpu.SemaphoreType.DMA,
            pltpu.SemaphoreType.DMA,
        ],
    )
    def ek(xi_hbm, ee_hbm, src_hbm, dst_hbm, zero_hbm, out_hbm,
           isrc, idst, eb, gb, acc, seme, semg):
        cid = lax.axis_index("c")
        sid = lax.axis_index("s")
        wid = sid * _NC + cid
        row0 = sid * _RT
        base = wid * _EW

        pltpu.sync_copy(zero_hbm.at[pl.ds(row0, _RT)], acc.at[pl.ds(row0, _RT)])
        plsc.subcore_barrier()

        def body(i, carry):
            off = base + i * _C
            pltpu.sync_copy(src_hbm.at[pl.ds(off, _C)], isrc)
            pltpu.sync_copy(dst_hbm.at[pl.ds(off, _C)], idst)
            de = pltpu.async_copy(ee_hbm.at[pl.ds(off, _C)], eb, seme)
            dg = pltpu.async_copy(xi_hbm.at[isrc], gb, semg)
            de.wait()
            dg.wait()

            def crow(r, c):
                for j in range(_H // 16):
                    sl = pl.ds(j * 16, 16)
                    gb[r, sl] = jnp.maximum(gb[r, sl] + eb[r, sl], 0.0)
                return c

            lax.fori_loop(0, _C, crow, 0)
            pltpu.sync_copy(gb, acc.at[idst], add=True)
            return carry

        lax.fori_loop(0, _NCHUNK, body, 0)
        plsc.subcore_barrier()
        pltpu.sync_copy(acc.at[pl.ds(row0, _RT)],
                        out_hbm.at[pl.ds(cid * _NP + row0, _RT)])

    return ek


def _sc_edge(xi, ee, src, dst, zeros):
    return _sc_edge_kernel()(xi, ee, src, dst, zeros)


def _dot(a, b):
    return jnp.dot(a.astype(jnp.bfloat16), b.astype(jnp.bfloat16),
                   preferred_element_type=jnp.float32)


def _dot_hi(a, b):
    return jnp.dot(a, b, preferred_element_type=jnp.float32,
                   precision=lax.Precision.HIGHEST)


def _bn(t, g, b):
    mu = jnp.mean(t, axis=0, keepdims=True)
    var = jnp.mean((t - mu) ** 2, axis=0, keepdims=True)
    return g * (t - mu) * jax.lax.rsqrt(var + 1e-5) + b


def _tc_call(body, out_shapes, *args):
    return pl.pallas_call(
        body,
        out_shape=out_shapes,
    )(*args)


def _k_in_body(x, w0, b0, g0, bb0, w1, b1, g1, bb1, w2, b2, vemb, x1_out):
    t = _dot(x[...], w0[...]) + b0[...]
    t = jax.nn.relu(_bn(t, g0[...], bb0[...]))
    t = _dot(t, w1[...]) + b1[...]
    t = jax.nn.relu(_bn(t, g1[...], bb1[...]))
    t = _dot(t, w2[...]) + b2[...]
    x1_out[...] = t + vemb[...]


_BE = 4000


def _k_ee_body(ea, w, b, ee_out):
    ee_out[...] = _dot(ea[...], w[...]) + b[...]


def _k_ee(ea, w, b):
    grid = _E // _BE
    return pl.pallas_call(
        _k_ee_body,
        grid=(grid,),
        in_specs=[
            pl.BlockSpec((_BE, _EDGE_DIM), lambda i: (i, 0)),
            pl.BlockSpec((_EDGE_DIM, _H), lambda i: (0, 0)),
            pl.BlockSpec((1, _H), lambda i: (0, 0)),
        ],
        out_specs=pl.BlockSpec((_BE, _H), lambda i: (i, 0)),
        out_shape=jax.ShapeDtypeStruct((_E, _H), jnp.float32),
    )(ea, w, b)


def _k_layer_body(xi, parts, ve, bcol, brow, eps,
                  w1, b1, g1, bb1, w2, b2,
                  vw1, vb1, vg1, vbb1, vw2, vb2, vg2, vbb2, vw3, vb3,
                  h_out, xn_out, ve_out):
    agg = parts[0:_N, :] + parts[_NP:_NP + _N, :]
    t = (1.0 + eps[0, 0]) * xi[...] + agg
    t = _dot(t, w1[...]) + b1[...]
    t = jax.nn.relu(_bn(t, g1[...], bb1[...]))
    t = _dot(t, w2[...]) + b2[...]
    h = t + xi[...]
    h_out[...] = h
    oht = (lax.broadcasted_iota(jnp.int32, (_G, _N), 0) == brow[...]
           ).astype(jnp.float32)
    pooled = _dot_hi(oht, h)
    v = pooled + ve[...]
    v = _dot(v, vw1[...]) + vb1[...]
    v = jax.nn.relu(_bn(v, vg1[...], vbb1[...]))
    v = _dot(v, vw2[...]) + vb2[...]
    v = jax.nn.relu(_bn(v, vg2[...], vbb2[...]))
    v = _dot(v, vw3[...]) + vb3[...]
    ve_new = ve[...] + v
    ve_out[...] = ve_new
    oh = (lax.broadcasted_iota(jnp.int32, (_N, _G), 1) == bcol[...]
          ).astype(jnp.float32)
    xn_out[...] = h + _dot_hi(oh, ve_new)


_BJ = 2000


def _k_join_body(x0, x1, x2, x3, x4, wa, wb, wc, wd, we, b1, y_out):
    y_out[...] = (_dot(x0[...], wa[...])
                  + _dot(x1[...], wb[...])
                  + _dot(x2[...], wc[...])
                  + _dot(x3[...], wd[...])
                  + _dot(x4[...], we[...])
                  + b1[...])


def _k_join(x0, x1, x2, x3, x4, wa, wb, wc, wd, we, b1):
    bs = lambda i: (i, 0)
    return pl.pallas_call(
        _k_join_body,
        grid=(_N // _BJ,),
        in_specs=[pl.BlockSpec((_BJ, _H), bs)] * 5
        + [pl.BlockSpec((_H, _H), lambda i: (0, 0))] * 5
        + [pl.BlockSpec((1, _H), lambda i: (0, 0))],
        out_specs=pl.BlockSpec((_BJ, _H), bs),
        out_shape=jax.ShapeDtypeStruct((_N, _H), jnp.float32),
    )(x0, x1, x2, x3, x4, wa, wb, wc, wd, we, b1)


def _k_out_body(y_in, g1, bb1, w2, b2, g2, bb2, w3, b3, brow,
                nodes_out, graph_out):
    y = jax.nn.relu(_bn(y_in[...], g1[...], bb1[...]))
    y = _dot(y, w2[...]) + b2[...]
    y = jax.nn.relu(_bn(y, g2[...], bb2[...]))
    nodes = _dot(y, w3[...]) + b3[...]
    nodes_out[...] = nodes
    oht = (lax.broadcasted_iota(jnp.int32, (_G, _N), 0) == brow[...]
           ).astype(jnp.float32)
    counts = jnp.maximum(jnp.sum(oht, axis=1, keepdims=True), 1.0)
    graph_out[...] = _dot_hi(oht, nodes) / counts


def _lin_t(p):
    return p["W"].T, p["b"].reshape(1, -1)


def kernel(x, edge_attr, params, edge_index, batch):
    src = edge_index[0]
    dst = edge_index[1]
    bcol = batch.reshape(_N, 1)
    brow = batch.reshape(1, _N)
    zeros = jnp.zeros((_NP, _H), jnp.float32)

    ip = params["in_layer"]
    w0, b0 = _lin_t(ip["lins"][0])
    w1, b1 = _lin_t(ip["lins"][1])
    w2, b2 = _lin_t(ip["lins"][2])
    g0 = ip["bns"][0]["g"].reshape(1, -1)
    bb0 = ip["bns"][0]["b"].reshape(1, -1)
    g1 = ip["bns"][1]["g"].reshape(1, -1)
    bb1 = ip["bns"][1]["b"].reshape(1, -1)
    vemb = params["virtual_embed"].reshape(1, _H)

    x1 = _tc_call(_k_in_body, jax.ShapeDtypeStruct((_N, _H), jnp.float32),
                  x, w0, b0, g0, bb0, w1, b1, g1, bb1, w2, b2, vemb)

    ve = jnp.tile(params["virtual_embed"], (_G, 1))
    xs = [x, x1]
    xi = x1
    for i in range(_DEPTH):
        cp = params["convs"][i]
        vp = params["virtual_layers"][i]
        we_w, we_b = _lin_t(cp["edge"])
        ee = _k_ee(edge_attr, we_w, we_b)
        parts = _sc_edge(xi, ee, src, dst, zeros)

        nw1, nb1 = _lin_t(cp["nn_lin1"])
        nw2, nb2 = _lin_t(cp["nn_lin2"])
        ng = cp["nn_bn"]["g"].reshape(1, -1)
        nbb = cp["nn_bn"]["b"].reshape(1, -1)
        vw1, vb1 = _lin_t(vp["lin1"])
        vw2, vb2 = _lin_t(vp["lin2"])
        vw3, vb3 = _lin_t(vp["lin3"])
        vg1 = vp["bn1"]["g"].reshape(1, -1)
        vbb1 = vp["bn1"]["b"].reshape(1, -1)
        vg2 = vp["bn2"]["g"].reshape(1, -1)
        vbb2 = vp["bn2"]["b"].reshape(1, -1)
        eps = cp["eps"].reshape(1, 1)

        h, xn, ve = _tc_call(
            _k_layer_body,
            (jax.ShapeDtypeStruct((_N, _H), jnp.float32),
             jax.ShapeDtypeStruct((_N, _H), jnp.float32),
             jax.ShapeDtypeStruct((_G, _H), jnp.float32)),
            xi, parts, ve, bcol, brow, eps,
            nw1, nb1, ng, nbb, nw2, nb2,
            vw1, vb1, vg1, vbb1, vw2, vb2, vg2, vbb2, vw3, vb3)
        if i < _DEPTH - 1:
            xs.append(xn)
            xi = xn
        else:
            xs.append(h)

    op = params["out_layer"]
    ow = op["lins"][0]["W"]
    ob1 = op["lins"][0]["b"].reshape(1, -1)
    wa = ow[:, 0:_H].T
    wb = ow[:, _H:2 * _H].T
    wc = ow[:, 2 * _H:3 * _H].T
    wd = ow[:, 3 * _H:4 * _H].T
    we = ow[:, 4 * _H:5 * _H].T
    ow2, ob2 = _lin_t(op["lins"][1])
    ow3, ob3 = _lin_t(op["lins"][2])
    og1 = op["bns"][0]["g"].reshape(1, -1)
    obb1 = op["bns"][0]["b"].reshape(1, -1)
    og2 = op["bns"][1]["g"].reshape(1, -1)
    obb2 = op["bns"][1]["b"].reshape(1, -1)

    y = _k_join(xs[0], xs[1], xs[2], xs[3], xs[4], wa, wb, wc, wd, we, ob1)
    nodes, graph = _tc_call(
        _k_out_body,
        (jax.ShapeDtypeStruct((_N, _H), jnp.float32),
         jax.ShapeDtypeStruct((_G, _H), jnp.float32)),
        y, og1, obb1, ow2, ob2, og2, obb2, ow3, ob3, brow)
    return (nodes, graph)


# speedup vs baseline: 3.0179x; 1.0153x over previous
import functools

import jax
import jax.numpy as jnp
from jax import lax
from jax.experimental import pallas as pl
from jax.experimental.pallas import tpu as pltpu
from jax.experimental.pallas import tpu_sc as plsc

_N = 10000
_E = 320000
_H = 128
_G = 128
_EDGE_DIM = 16
_DEPTH = 3

_NC = 2
_NS = 16
_NW = _NC * _NS
_EW = _E // _NW
_C = 40
_NCHUNK = _EW // _C
_NP = 10240
_RT = _NP // _NS


def _sc_mesh():
    return plsc.VectorSubcoreMesh(core_axis_name="c", subcore_axis_name="s",
                                  num_cores=_NC, num_subcores=_NS)


@functools.cache
def _sc_edge_kernel():
    @functools.partial(
        pl.kernel,
        out_type=jax.ShapeDtypeStruct((2 * _NP, _H), jnp.float32),
        mesh=_sc_mesh(),
        scratch_types=[
            pltpu.VMEM((_C,), jnp.int32),
            pltpu.VMEM((_C,), jnp.int32),
            pltpu.VMEM((_C,), jnp.int32),
            pltpu.VMEM((_C,), jnp.int32),
            pltpu.VMEM((_C, _H), jnp.float32),
            pltpu.VMEM((_C, _H), jnp.float32),
            pltpu.VMEM((_C, _H), jnp.float32),
            pltpu.VMEM((_C, _H), jnp.float32),
            pltpu.VMEM_SHARED((_NP, _H), jnp.float32),
            pltpu.SemaphoreType.DMA,
            pltpu.SemaphoreType.DMA,
            pltpu.SemaphoreType.DMA,
            pltpu.SemaphoreType.DMA,
        ],
    )
    def ek(xi_hbm, ee_hbm, src_hbm, dst_hbm, zero_hbm, out_hbm,
           isrcA, idstA, isrcB, idstB, ebA, gbA, ebB, gbB, acc,
           semeA, semgA, semeB, semgB):
        cid = lax.axis_index("c")
        sid = lax.axis_index("s")
        wid = sid * _NC + cid
        row0 = sid * _RT
        base = wid * _EW

        pltpu.sync_copy(zero_hbm.at[pl.ds(row0, _RT)], acc.at[pl.ds(row0, _RT)])
        plsc.subcore_barrier()

        def compute(eb, gb):
            def crow(r, c):
                for j in range(_H // 16):
                    sl = pl.ds(j * 16, 16)
                    gb[r, sl] = jnp.maximum(gb[r, sl] + eb[r, sl], 0.0)
                return c

            lax.fori_loop(0, _C, crow, 0)

        def body(p, carry):
            offA = base + (2 * p) * _C
            offB = offA + _C
            pltpu.sync_copy(src_hbm.at[pl.ds(offA, _C)], isrcA)
            pltpu.sync_copy(dst_hbm.at[pl.ds(offA, _C)], idstA)
            deA = pltpu.async_copy(ee_hbm.at[pl.ds(offA, _C)], ebA, semeA)
            dgA = pltpu.async_copy(xi_hbm.at[isrcA], gbA, semgA)
            pltpu.sync_copy(src_hbm.at[pl.ds(offB, _C)], isrcB)
            pltpu.sync_copy(dst_hbm.at[pl.ds(offB, _C)], idstB)
            deB = pltpu.async_copy(ee_hbm.at[pl.ds(offB, _C)], ebB, semeB)
            dgB = pltpu.async_copy(xi_hbm.at[isrcB], gbB, semgB)
            deA.wait()
            dgA.wait()
            compute(ebA, gbA)
            pltpu.sync_copy(gbA, acc.at[idstA], add=True)
            deB.wait()
            dgB.wait()
            compute(ebB, gbB)
            pltpu.sync_copy(gbB, acc.at[idstB], add=True)
            return carry

        lax.fori_loop(0, _NCHUNK // 2, body, 0)
        plsc.subcore_barrier()
        pltpu.sync_copy(acc.at[pl.ds(row0, _RT)],
                        out_hbm.at[pl.ds(cid * _NP + row0, _RT)])

    return ek


def _sc_edge(xi, ee, src, dst, zeros):
    return _sc_edge_kernel()(xi, ee, src, dst, zeros)


def _dot(a, b):
    return jnp.dot(a.astype(jnp.bfloat16), b.astype(jnp.bfloat16),
                   preferred_element_type=jnp.float32)


def _dot_hi(a, b):
    return jnp.dot(a, b, preferred_element_type=jnp.float32,
                   precision=lax.Precision.HIGHEST)


def _bn(t, g, b):
    mu = jnp.mean(t, axis=0, keepdims=True)
    var = jnp.mean((t - mu) ** 2, axis=0, keepdims=True)
    return g * (t - mu) * jax.lax.rsqrt(var + 1e-5) + b


def _tc_call(body, out_shapes, *args):
    return pl.pallas_call(
        body,
        out_shape=out_shapes,
    )(*args)


def _k_in_body(x, w0, b0, g0, bb0, w1, b1, g1, bb1, w2, b2, vemb, x1_out):
    t = _dot(x[...], w0[...]) + b0[...]
    t = jax.nn.relu(_bn(t, g0[...], bb0[...]))
    t = _dot(t, w1[...]) + b1[...]
    t = jax.nn.relu(_bn(t, g1[...], bb1[...]))
    t = _dot(t, w2[...]) + b2[...]
    x1_out[...] = t + vemb[...]


_BE = 4000


def _k_ee_body(ea, w, b, ee_out):
    ee_out[...] = _dot(ea[...], w[...]) + b[...]


def _k_ee(ea, w, b):
    grid = _E // _BE
    return pl.pallas_call(
        _k_ee_body,
        grid=(grid,),
        in_specs=[
            pl.BlockSpec((_BE, _EDGE_DIM), lambda i: (i, 0)),
            pl.BlockSpec((_EDGE_DIM, _H), lambda i: (0, 0)),
            pl.BlockSpec((1, _H), lambda i: (0, 0)),
        ],
        out_specs=pl.BlockSpec((_BE, _H), lambda i: (i, 0)),
        out_shape=jax.ShapeDtypeStruct((_E, _H), jnp.float32),
    )(ea, w, b)


def _k_layer_body(xi, parts, ve, bcol, brow, eps,
                  w1, b1, g1, bb1, w2, b2,
                  vw1, vb1, vg1, vbb1, vw2, vb2, vg2, vbb2, vw3, vb3,
                  h_out, xn_out, ve_out):
    agg = parts[0:_N, :] + parts[_NP:_NP + _N, :]
    t = (1.0 + eps[0, 0]) * xi[...] + agg
    t = _dot(t, w1[...]) + b1[...]
    t = jax.nn.relu(_bn(t, g1[...], bb1[...]))
    t = _dot(t, w2[...]) + b2[...]
    h = t + xi[...]
    h_out[...] = h
    oht = (lax.broadcasted_iota(jnp.int32, (_G, _N), 0) == brow[...]
           ).astype(jnp.float32)
    pooled = _dot_hi(oht, h)
    v = pooled + ve[...]
    v = _dot(v, vw1[...]) + vb1[...]
    v = jax.nn.relu(_bn(v, vg1[...], vbb1[...]))
    v = _dot(v, vw2[...]) + vb2[...]
    v = jax.nn.relu(_bn(v, vg2[...], vbb2[...]))
    v = _dot(v, vw3[...]) + vb3[...]
    ve_new = ve[...] + v
    ve_out[...] = ve_new
    oh = (lax.broadcasted_iota(jnp.int32, (_N, _G), 1) == bcol[...]
          ).astype(jnp.float32)
    xn_out[...] = h + _dot_hi(oh, ve_new)


_BJ = 2000


def _k_join_body(x0, x1, x2, x3, x4, wa, wb, wc, wd, we, b1, y_out):
    y_out[...] = (_dot(x0[...], wa[...])
                  + _dot(x1[...], wb[...])
                  + _dot(x2[...], wc[...])
                  + _dot(x3[...], wd[...])
                  + _dot(x4[...], we[...])
                  + b1[...])


def _k_join(x0, x1, x2, x3, x4, wa, wb, wc, wd, we, b1):
    bs = lambda i: (i, 0)
    return pl.pallas_call(
        _k_join_body,
        grid=(_N // _BJ,),
        in_specs=[pl.BlockSpec((_BJ, _H), bs)] * 5
        + [pl.BlockSpec((_H, _H), lambda i: (0, 0))] * 5
        + [pl.BlockSpec((1, _H), lambda i: (0, 0))],
        out_specs=pl.BlockSpec((_BJ, _H), bs),
        out_shape=jax.ShapeDtypeStruct((_N, _H), jnp.float32),
    )(x0, x1, x2, x3, x4, wa, wb, wc, wd, we, b1)


def _k_out_body(y_in, g1, bb1, w2, b2, g2, bb2, w3, b3, brow,
                nodes_out, graph_out):
    y = jax.nn.relu(_bn(y_in[...], g1[...], bb1[...]))
    y = _dot(y, w2[...]) + b2[...]
    y = jax.nn.relu(_bn(y, g2[...], bb2[...]))
    nodes = _dot(y, w3[...]) + b3[...]
    nodes_out[...] = nodes
    oht = (lax.broadcasted_iota(jnp.int32, (_G, _N), 0) == brow[...]
           ).astype(jnp.float32)
    counts = jnp.maximum(jnp.sum(oht, axis=1, keepdims=True), 1.0)
    graph_out[...] = _dot_hi(oht, nodes) / counts


def _lin_t(p):
    return p["W"].T, p["b"].reshape(1, -1)


def kernel(x, edge_attr, params, edge_index, batch):
    src = edge_index[0]
    dst = edge_index[1]
    bcol = batch.reshape(_N, 1)
    brow = batch.reshape(1, _N)
    zeros = jnp.zeros((_NP, _H), jnp.float32)

    ip = params["in_layer"]
    w0, b0 = _lin_t(ip["lins"][0])
    w1, b1 = _lin_t(ip["lins"][1])
    w2, b2 = _lin_t(ip["lins"][2])
    g0 = ip["bns"][0]["g"].reshape(1, -1)
    bb0 = ip["bns"][0]["b"].reshape(1, -1)
    g1 = ip["bns"][1]["g"].reshape(1, -1)
    bb1 = ip["bns"][1]["b"].reshape(1, -1)
    vemb = params["virtual_embed"].reshape(1, _H)

    x1 = _tc_call(_k_in_body, jax.ShapeDtypeStruct((_N, _H), jnp.float32),
                  x, w0, b0, g0, bb0, w1, b1, g1, bb1, w2, b2, vemb)

    ve = jnp.tile(params["virtual_embed"], (_G, 1))
    xs = [x, x1]
    xi = x1
    for i in range(_DEPTH):
        cp = params["convs"][i]
        vp = params["virtual_layers"][i]
        we_w, we_b = _lin_t(cp["edge"])
        ee = _k_ee(edge_attr, we_w, we_b)
        parts = _sc_edge(xi, ee, src, dst, zeros)

        nw1, nb1 = _lin_t(cp["nn_lin1"])
        nw2, nb2 = _lin_t(cp["nn_lin2"])
        ng = cp["nn_bn"]["g"].reshape(1, -1)
        nbb = cp["nn_bn"]["b"].reshape(1, -1)
        vw1, vb1 = _lin_t(vp["lin1"])
        vw2, vb2 = _lin_t(vp["lin2"])
        vw3, vb3 = _lin_t(vp["lin3"])
        vg1 = vp["bn1"]["g"].reshape(1, -1)
        vbb1 = vp["bn1"]["b"].reshape(1, -1)
        vg2 = vp["bn2"]["g"].reshape(1, -1)
        vbb2 = vp["bn2"]["b"].reshape(1, -1)
        eps = cp["eps"].reshape(1, 1)

        h, xn, ve = _tc_call(
            _k_layer_body,
            (jax.ShapeDtypeStruct((_N, _H), jnp.float32),
             jax.ShapeDtypeStruct((_N, _H), jnp.float32),
             jax.ShapeDtypeStruct((_G, _H), jnp.float32)),
            xi, parts, ve, bcol, brow, eps,
            nw1, nb1, ng, nbb, nw2, nb2,
            vw1, vb1, vg1, vbb1, vw2, vb2, vg2, vbb2, vw3, vb3)
        if i < _DEPTH - 1:
            xs.append(xn)
            xi = xn
        else:
            xs.append(h)

    op = params["out_layer"]
    ow = op["lins"][0]["W"]
    ob1 = op["lins"][0]["b"].reshape(1, -1)
    wa = ow[:, 0:_H].T
    wb = ow[:, _H:2 * _H].T
    wc = ow[:, 2 * _H:3 * _H].T
    wd = ow[:, 3 * _H:4 * _H].T
    we = ow[:, 4 * _H:5 * _H].T
    ow2, ob2 = _lin_t(op["lins"][1])
    ow3, ob3 = _lin_t(op["lins"][2])
    og1 = op["bns"][0]["g"].reshape(1, -1)
    obb1 = op["bns"][0]["b"].reshape(1, -1)
    og2 = op["bns"][1]["g"].reshape(1, -1)
    obb2 = op["bns"][1]["b"].reshape(1, -1)

    y = _k_join(xs[0], xs[1], xs[2], xs[3], xs[4], wa, wb, wc, wd, we, ob1)
    nodes, graph = _tc_call(
        _k_out_body,
        (jax.ShapeDtypeStruct((_N, _H), jnp.float32),
         jax.ShapeDtypeStruct((_G, _H), jnp.float32)),
        y, og1, obb1, ow2, ob2, og2, obb2, ow3, ob3, brow)
    return (nodes, graph)
